# baseline (speedup 1.0000x reference)
import jax
import jax.numpy as jnp
from jax import lax
from jax.experimental import pallas as pl
from jax.experimental.pallas import tpu as pltpu
from jax.experimental.pallas import tpu_sc as plsc

T0, T1, D = 512, 512, 64
NC, NS, L = 2, 16, 16
NW = NC * NS
K = 2
C = K * 128
G = 128
NI, NJ = 4096, 200
NIT = NI // 128
NB2 = NJ * NIT // K
PER_W = NB2 // NW
R = 4


def _sc_embed_body(xf_hbm, tab_hbm, out_hbm, xv0, xv1, xv2, xv3, idx0, idx1,
                   idx2, idx3, rows0, rows1, rows2, rows3, tr0, tr1, sem_x,
                   sem_g, sem_o):
    wid = lax.axis_index("s") * NC + lax.axis_index("c")
    lane = lax.iota(jnp.int32, L)
    xvs, idxs = (xv0, xv1, xv2, xv3), (idx0, idx1, idx2, idx3)
    rowss, trs = (rows0, rows1, rows2, rows3), (tr0, tr1)
    base = wid * PER_W

    def quantize(v):
        t = (v * 0.5 + 0.5) * 512.0
        t = jnp.minimum(t, 511.0)
        t = jnp.maximum(t, 0.0)
        return t.astype(jnp.int32)

    def fire_x(b, xv):
        bid2 = jnp.minimum(base + b, NB2 - 1)
        pltpu.async_copy(xf_hbm.at[pl.ds(bid2 * (2 * C), 2 * C)], xv, sem_x)

    def wait_x():
        pltpu.make_async_copy(xf_hbm.at[pl.ds(0, 2 * C)], xv0, sem_x).wait()

    def comp_idx(xv, idx_v):
        @plsc.parallel_loop(0, C // L, 1, unroll=4)
        def comp(m):
            itp = m // (128 // L)
            mm = m % (128 // L)
            x0 = xv[pl.ds(itp * 256 + mm * L, L)]
            x1 = xv[pl.ds(itp * 256 + 128 + mm * L, L)]
            idx_v[pl.ds(m * L, L)] = quantize(x0) * 512 + quantize(x1)

    def fire_gathers(idx_v, rows):
        for g in range(C // G):
            pltpu.async_copy(
                tab_hbm.at[idx_v.at[pl.ds(g * G, G)]],
                rows.at[pl.ds(g * G, G)],
                sem_g,
            )

    def wait_gathers():
        for g in range(C // G):
            pltpu.make_async_copy(
                tab_hbm.at[idx0.at[pl.ds(0, G)]], rows0.at[pl.ds(0, G)], sem_g
            ).wait()

    def transpose(rows, tr):
        @plsc.parallel_loop(0, 64 * (C // L), 1, unroll=2)
        def tbody(u):
            k = u // (C // L)
            m = u % (C // L)
            t0 = m * L
            v = plsc.load_gather(rows, [t0 + lane, jnp.broadcast_to(k, (L,))])
            inner = (t0 // 128) * 1024 + (k % 8) * 128 + (t0 % 128)
            tr[k // 8, pl.ds(inner, L)] = v

    def fire_out(b, tr):
        bid2 = base + b
        j = bid2 // (NIT // K)
        itp = bid2 % (NIT // K)
        pltpu.async_copy(tr, out_hbm.at[j, :, itp, :], sem_o)

    def wait_out():
        pltpu.make_async_copy(tr0, out_hbm.at[0, :, 0, :], sem_o).wait()

    fire_x(0, xvs[0])
    fire_x(1, xvs[1])
    wait_x()
    comp_idx(xvs[0], idxs[0])
    fire_gathers(idxs[0], rowss[0])
    fire_x(2, xvs[2])
    wait_x()
    comp_idx(xvs[1], idxs[1])
    fire_gathers(idxs[1], rowss[1])
    fire_x(3, xvs[3])
    wait_x()
    comp_idx(xvs[2], idxs[2])
    fire_gathers(idxs[2], rowss[2])
    wait_gathers()
    transpose(rowss[0], trs[0])
    fire_out(0, trs[0])
    fire_x(4, xvs[0])
    wait_x()
    comp_idx(xvs[3], idxs[3])
    fire_gathers(idxs[3], rowss[3])
    wait_gathers()
    transpose(rowss[1], trs[1])
    fire_out(1, trs[1])

    def body4(kk, carry):
        for q in range(R):
            b = 4 + R * kk + q
            fire_x(b + 1, xvs[(q + 1) % R])
            wait_x()
            comp_idx(xvs[q], idxs[q])
            fire_gathers(idxs[q], rowss[q])
            wait_gathers()
            wait_out()
            transpose(rowss[(q + 2) % R], trs[q % 2])
            fire_out(b - 2, trs[q % 2])
        return carry

    lax.fori_loop(0, (PER_W - 4) // R, body4, 0)

    wait_x()
    wait_gathers()
    wait_out()
    transpose(rowss[98 % R], trs[0])
    fire_out(98, trs[0])
    wait_gathers()
    wait_out()
    transpose(rowss[99 % R], trs[1])
    fire_out(99, trs[1])
    wait_out()
    wait_out()


@jax.jit
def _sc_embed(xf, tab):
    mesh = plsc.VectorSubcoreMesh(core_axis_name="c", subcore_axis_name="s")
    return pl.kernel(
        _sc_embed_body,
        out_type=jax.ShapeDtypeStruct((NJ, 8, NIT // K, K * 1024),
                                      jnp.float32),
        mesh=mesh,
        compiler_params=pltpu.CompilerParams(
            needs_layout_passes=False, use_tc_tiling_on_sc=False
        ),
        scratch_types=[
            pltpu.VMEM((2 * C,), jnp.float32),
            pltpu.VMEM((2 * C,), jnp.float32),
            pltpu.VMEM((2 * C,), jnp.float32),
            pltpu.VMEM((2 * C,), jnp.float32),
            pltpu.VMEM((C,), jnp.int32),
            pltpu.VMEM((C,), jnp.int32),
            pltpu.VMEM((C,), jnp.int32),
            pltpu.VMEM((C,), jnp.int32),
            pltpu.VMEM((C, D), jnp.float32),
            pltpu.VMEM((C, D), jnp.float32),
            pltpu.VMEM((C, D), jnp.float32),
            pltpu.VMEM((C, D), jnp.float32),
            pltpu.VMEM((8, K * 1024), jnp.float32),
            pltpu.VMEM((8, K * 1024), jnp.float32),
            pltpu.SemaphoreType.DMA,
            pltpu.SemaphoreType.DMA,
            pltpu.SemaphoreType.DMA,
        ],
    )(xf, tab)


def kernel(x, table):
    xf = x.reshape(NIT, 128, NJ, 2).transpose(2, 0, 3, 1).reshape(-1)
    tab = table.reshape(T0 * T1, D)
    out = _sc_embed(xf, tab)
    return (
        out.reshape(NJ, 8, NIT, 8, 128)
        .transpose(2, 4, 0, 1, 3)
        .reshape(NI, NJ, D)
    )


# speedup vs baseline: 1.1810x; 1.1810x over previous
import jax
import jax.numpy as jnp
from jax import lax
from jax.experimental import pallas as pl
from jax.experimental.pallas import tpu as pltpu
from jax.experimental.pallas import tpu_sc as plsc

T0, T1, D = 512, 512, 64
NC, NS, L = 2, 16, 16
NW = NC * NS
K = 2
C = K * 128
G = 128
NI, NJ = 4096, 200
NIT = NI // 128
NB2 = NJ * NIT // K
PER_W = NB2 // NW
R = 4


def _sc_embed_body(xf_hbm, tab_hbm, out_hbm, xv0, xv1, xv2, xv3, idx0, idx1,
                   idx2, idx3, rows0, rows1, rows2, rows3, tr0, tr1, sem_x,
                   sem_g, sem_o):
    wid = lax.axis_index("s") * NC + lax.axis_index("c")
    lane = lax.iota(jnp.int32, L)
    xvs, idxs = (xv0, xv1, xv2, xv3), (idx0, idx1, idx2, idx3)
    rowss, trs = (rows0, rows1, rows2, rows3), (tr0, tr1)
    base = wid * PER_W

    def quantize(v):
        t = (v * 0.5 + 0.5) * 512.0
        t = jnp.minimum(t, 511.0)
        t = jnp.maximum(t, 0.0)
        return t.astype(jnp.int32)

    def fire_x(b, xv):
        bid2 = jnp.minimum(base + b, NB2 - 1)
        pltpu.async_copy(xf_hbm.at[pl.ds(bid2 * (2 * C), 2 * C)], xv, sem_x)

    def wait_x():
        pltpu.make_async_copy(xf_hbm.at[pl.ds(0, 2 * C)], xv0, sem_x).wait()

    def comp_idx(xv, idx_v):
        @plsc.parallel_loop(0, C // L, 1, unroll=4)
        def comp(m):
            itp = m // (128 // L)
            mm = m % (128 // L)
            x0 = xv[pl.ds(itp * 256 + mm * L, L)]
            x1 = xv[pl.ds(itp * 256 + 128 + mm * L, L)]
            idx_v[pl.ds(m * L, L)] = quantize(x0) * 512 + quantize(x1)

    def fire_gathers(idx_v, rows):
        for g in range(C // G):
            pltpu.async_copy(
                tab_hbm.at[idx_v.at[pl.ds(g * G, G)]],
                rows.at[pl.ds(g * G, G)],
                sem_g,
            )

    def wait_gathers():
        for g in range(C // G):
            pltpu.make_async_copy(
                tab_hbm.at[idx0.at[pl.ds(0, G)]], rows0.at[pl.ds(0, G)], sem_g
            ).wait()

    def transpose(rows, tr):
        @plsc.parallel_loop(0, 64 * (C // L), 1, unroll=16)
        def tbody(u):
            k = u // (C // L)
            m = u % (C // L)
            t0 = m * L
            v = plsc.load_gather(rows, [t0 + lane, jnp.broadcast_to(k, (L,))])
            inner = (t0 // 128) * 1024 + (k % 8) * 128 + (t0 % 128)
            tr[k // 8, pl.ds(inner, L)] = v

    def fire_out(b, tr):
        bid2 = base + b
        j = bid2 // (NIT // K)
        itp = bid2 % (NIT // K)
        pltpu.async_copy(tr, out_hbm.at[j, :, itp, :], sem_o)

    def wait_out():
        pltpu.make_async_copy(tr0, out_hbm.at[0, :, 0, :], sem_o).wait()

    fire_x(0, xvs[0])
    fire_x(1, xvs[1])
    wait_x()
    comp_idx(xvs[0], idxs[0])
    fire_gathers(idxs[0], rowss[0])
    fire_x(2, xvs[2])
    wait_x()
    comp_idx(xvs[1], idxs[1])
    fire_gathers(idxs[1], rowss[1])
    fire_x(3, xvs[3])
    wait_x()
    comp_idx(xvs[2], idxs[2])
    fire_gathers(idxs[2], rowss[2])
    wait_gathers()
    transpose(rowss[0], trs[0])
    fire_out(0, trs[0])
    fire_x(4, xvs[0])
    wait_x()
    comp_idx(xvs[3], idxs[3])
    fire_gathers(idxs[3], rowss[3])
    wait_gathers()
    transpose(rowss[1], trs[1])
    fire_out(1, trs[1])

    def body4(kk, carry):
        for q in range(R):
            b = 4 + R * kk + q
            fire_x(b + 1, xvs[(q + 1) % R])
            wait_x()
            comp_idx(xvs[q], idxs[q])
            fire_gathers(idxs[q], rowss[q])
            wait_gathers()
            wait_out()
            transpose(rowss[(q + 2) % R], trs[q % 2])
            fire_out(b - 2, trs[q % 2])
        return carry

    lax.fori_loop(0, (PER_W - 4) // R, body4, 0)

    wait_x()
    wait_gathers()
    wait_out()
    transpose(rowss[98 % R], trs[0])
    fire_out(98, trs[0])
    wait_gathers()
    wait_out()
    transpose(rowss[99 % R], trs[1])
    fire_out(99, trs[1])
    wait_out()
    wait_out()


@jax.jit
def _sc_embed(xf, tab):
    mesh = plsc.VectorSubcoreMesh(core_axis_name="c", subcore_axis_name="s")
    return pl.kernel(
        _sc_embed_body,
        out_type=jax.ShapeDtypeStruct((NJ, 8, NIT // K, K * 1024),
                                      jnp.float32),
        mesh=mesh,
        compiler_params=pltpu.CompilerParams(
            needs_layout_passes=False, use_tc_tiling_on_sc=False
        ),
        scratch_types=[
            pltpu.VMEM((2 * C,), jnp.float32),
            pltpu.VMEM((2 * C,), jnp.float32),
            pltpu.VMEM((2 * C,), jnp.float32),
            pltpu.VMEM((2 * C,), jnp.float32),
            pltpu.VMEM((C,), jnp.int32),
            pltpu.VMEM((C,), jnp.int32),
            pltpu.VMEM((C,), jnp.int32),
            pltpu.VMEM((C,), jnp.int32),
            pltpu.VMEM((C, D), jnp.float32),
            pltpu.VMEM((C, D), jnp.float32),
            pltpu.VMEM((C, D), jnp.float32),
            pltpu.VMEM((C, D), jnp.float32),
            pltpu.VMEM((8, K * 1024), jnp.float32),
            pltpu.VMEM((8, K * 1024), jnp.float32),
            pltpu.SemaphoreType.DMA,
            pltpu.SemaphoreType.DMA,
            pltpu.SemaphoreType.DMA,
        ],
    )(xf, tab)


def kernel(x, table):
    xf = x.reshape(NIT, 128, NJ, 2).transpose(2, 0, 3, 1).reshape(-1)
    tab = table.reshape(T0 * T1, D)
    out = _sc_embed(xf, tab)
    return (
        out.reshape(NJ, 8, NIT, 8, 128)
        .transpose(2, 4, 0, 1, 3)
        .reshape(NI, NJ, D)
    )
